# baseline (speedup 1.0000x reference)
import functools

import jax
import jax.numpy as jnp
from jax import lax
from jax.experimental import pallas as pl
from jax.experimental.pallas import tpu as pltpu
from jax.experimental.pallas import tpu_sc as plsc

SEQ_LEN = 8192
D_MODEL = 1024
LANES = 16
NUM_WORKERS = 32

SC_ROWS = 2048
TC_ROWS = SEQ_LEN - SC_ROWS
ROWS_PER_WORKER = SC_ROWS // NUM_WORKERS
CHUNK_ROWS = 16
NUM_CHUNKS = ROWS_PER_WORKER // CHUNK_ROWS

TC_BLOCK_ROWS = 512

_mesh = plsc.VectorSubcoreMesh(core_axis_name="c", subcore_axis_name="s")

_CHUNK = (CHUNK_ROWS, D_MODEL)


@functools.partial(
    pl.kernel,
    mesh=_mesh,
    out_type=jax.ShapeDtypeStruct((SC_ROWS, D_MODEL), jnp.float32),
    scratch_types=[
        pltpu.VMEM(_CHUNK, jnp.float32),
        pltpu.VMEM(_CHUNK, jnp.float32),
        pltpu.VMEM(_CHUNK, jnp.float32),
        pltpu.VMEM(_CHUNK, jnp.float32),
        pltpu.VMEM(_CHUNK, jnp.float32),
        pltpu.VMEM(_CHUNK, jnp.float32),
        pltpu.SemaphoreType.DMA,
        pltpu.SemaphoreType.DMA,
        pltpu.SemaphoreType.DMA,
        pltpu.SemaphoreType.DMA,
        pltpu.SemaphoreType.DMA,
        pltpu.SemaphoreType.DMA,
    ],
)
def _sc_add(x_hbm, emb_hbm, out_hbm,
            xb0, xb1, eb0, eb1, ob0, ob1,
            sx0, sx1, se0, se1, so0, so1):
    xbufs = (xb0, xb1)
    ebufs = (eb0, eb1)
    obufs = (ob0, ob1)
    sxs = (sx0, sx1)
    ses = (se0, se1)
    sos = (so0, so1)

    wid = lax.axis_index("s") * 2 + lax.axis_index("c")
    base = wid * ROWS_PER_WORKER

    def rows_at(ci):
        return pl.ds(base + ci * CHUNK_ROWS, CHUNK_ROWS)

    def start_gather(ci, b):
        pltpu.async_copy(x_hbm.at[rows_at(ci), :], xbufs[b], sxs[b])
        pltpu.async_copy(emb_hbm.at[rows_at(ci), :], ebufs[b], ses[b])

    def wait_gather(b):
        pltpu.make_async_copy(x_hbm.at[rows_at(0), :], xbufs[b], sxs[b]).wait()
        pltpu.make_async_copy(emb_hbm.at[rows_at(0), :], ebufs[b], ses[b]).wait()

    def wait_scatter(b):
        pltpu.make_async_copy(obufs[b], out_hbm.at[rows_at(0), :], sos[b]).wait()

    start_gather(0, 0)

    def outer(g, carry):
        for b in (0, 1):
            ci = 2 * g + b
            @pl.when(ci + 1 < NUM_CHUNKS)
            def _():
                start_gather(ci + 1, 1 - b)

            wait_gather(b)

            @pl.when(ci >= 2)
            def _():
                wait_scatter(b)

            xbuf, ebuf, obuf = xbufs[b], ebufs[b], obufs[b]

            def row_body(r, rcarry):
                for j in range(D_MODEL // LANES):
                    sl = pl.ds(j * LANES, LANES)
                    obuf[r, sl] = xbuf[r, sl] + ebuf[r, sl]
                return rcarry

            lax.fori_loop(0, CHUNK_ROWS, row_body, 0)
            pltpu.async_copy(obuf, out_hbm.at[rows_at(ci), :], sos[b])
        return carry

    lax.fori_loop(0, NUM_CHUNKS // 2, outer, 0)
    wait_scatter(0)
    wait_scatter(1)


def _tc_body(x_ref, emb_ref, out_ref):
    out_ref[...] = x_ref[...] + emb_ref[...]


_TC_OFF = SC_ROWS // TC_BLOCK_ROWS


def _tc_add(x, emb):
    grid = (TC_ROWS // TC_BLOCK_ROWS,)
    in_spec = pl.BlockSpec((TC_BLOCK_ROWS, D_MODEL), lambda i: (i + _TC_OFF, 0))
    out_spec = pl.BlockSpec((TC_BLOCK_ROWS, D_MODEL), lambda i: (i, 0))
    return pl.pallas_call(
        _tc_body,
        grid=grid,
        in_specs=[in_spec, in_spec],
        out_specs=out_spec,
        out_shape=jax.ShapeDtypeStruct((TC_ROWS, D_MODEL), jnp.float32),
    )(x, emb)


def kernel(x, emb):
    sc_part = _sc_add(x, emb)
    tc_part = _tc_add(x, emb)
    return jnp.concatenate([sc_part, tc_part], axis=0)


# speedup vs baseline: 1.2303x; 1.2303x over previous
import functools

import jax
import jax.numpy as jnp
from jax import lax
from jax.experimental import pallas as pl
from jax.experimental.pallas import tpu as pltpu
from jax.experimental.pallas import tpu_sc as plsc

SEQ_LEN = 8192
D_MODEL = 1024
LANES = 16
NUM_WORKERS = 32

SC_ROWS = 4096
TC_ROWS = SEQ_LEN - SC_ROWS
ROWS_PER_WORKER = SC_ROWS // NUM_WORKERS
CHUNK_ROWS = 16
NUM_CHUNKS = ROWS_PER_WORKER // CHUNK_ROWS

TC_BLOCK_ROWS = 512
_TC_OFF = SC_ROWS // TC_BLOCK_ROWS

_mesh = plsc.VectorSubcoreMesh(core_axis_name="c", subcore_axis_name="s")

_CHUNK = (CHUNK_ROWS, D_MODEL)


@functools.partial(
    pl.kernel,
    mesh=_mesh,
    out_type=jax.ShapeDtypeStruct((SEQ_LEN, D_MODEL), jnp.float32),
    scratch_types=[
        pltpu.VMEM(_CHUNK, jnp.float32),
        pltpu.VMEM(_CHUNK, jnp.float32),
        pltpu.VMEM(_CHUNK, jnp.float32),
        pltpu.VMEM(_CHUNK, jnp.float32),
        pltpu.VMEM(_CHUNK, jnp.float32),
        pltpu.VMEM(_CHUNK, jnp.float32),
        pltpu.SemaphoreType.DMA,
        pltpu.SemaphoreType.DMA,
        pltpu.SemaphoreType.DMA,
        pltpu.SemaphoreType.DMA,
        pltpu.SemaphoreType.DMA,
        pltpu.SemaphoreType.DMA,
    ],
)
def _sc_add(x_hbm, emb_hbm, out_hbm,
            xb0, xb1, eb0, eb1, ob0, ob1,
            sx0, sx1, se0, se1, so0, so1):
    xbufs = (xb0, xb1)
    ebufs = (eb0, eb1)
    obufs = (ob0, ob1)
    sxs = (sx0, sx1)
    ses = (se0, se1)
    sos = (so0, so1)

    wid = lax.axis_index("s") * 2 + lax.axis_index("c")
    base = wid * ROWS_PER_WORKER

    def rows_at(ci):
        return pl.ds(base + ci * CHUNK_ROWS, CHUNK_ROWS)

    def start_gather(ci, b):
        pltpu.async_copy(x_hbm.at[rows_at(ci), :], xbufs[b], sxs[b])
        pltpu.async_copy(emb_hbm.at[rows_at(ci), :], ebufs[b], ses[b])

    def wait_gather(b):
        pltpu.make_async_copy(x_hbm.at[rows_at(0), :], xbufs[b], sxs[b]).wait()
        pltpu.make_async_copy(emb_hbm.at[rows_at(0), :], ebufs[b], ses[b]).wait()

    def wait_scatter(b):
        pltpu.make_async_copy(obufs[b], out_hbm.at[rows_at(0), :], sos[b]).wait()

    start_gather(0, 0)

    def outer(g, carry):
        for b in (0, 1):
            ci = 2 * g + b
            @pl.when(ci + 1 < NUM_CHUNKS)
            def _():
                start_gather(ci + 1, 1 - b)

            wait_gather(b)

            @pl.when(ci >= 2)
            def _():
                wait_scatter(b)

            xbuf, ebuf, obuf = xbufs[b], ebufs[b], obufs[b]

            def row_body(r, rcarry):
                for j in range(D_MODEL // LANES):
                    sl = pl.ds(j * LANES, LANES)
                    obuf[r, sl] = xbuf[r, sl] + ebuf[r, sl]
                return rcarry

            lax.fori_loop(0, CHUNK_ROWS, row_body, 0)
            pltpu.async_copy(obuf, out_hbm.at[rows_at(ci), :], sos[b])
        return carry

    lax.fori_loop(0, NUM_CHUNKS // 2, outer, 0)
    wait_scatter(0)
    wait_scatter(1)


def _tc_body(x_ref, emb_ref, prev_ref, out_ref):
    del prev_ref
    out_ref[...] = x_ref[...] + emb_ref[...]


def _tc_finish(x, emb, prev):
    grid = (TC_ROWS // TC_BLOCK_ROWS,)
    in_spec = pl.BlockSpec((TC_BLOCK_ROWS, D_MODEL), lambda i: (i + _TC_OFF, 0))
    return pl.pallas_call(
        _tc_body,
        grid=grid,
        in_specs=[
            in_spec,
            in_spec,
            pl.BlockSpec(memory_space=pltpu.MemorySpace.HBM),
        ],
        out_specs=pl.BlockSpec((TC_BLOCK_ROWS, D_MODEL), lambda i: (i + _TC_OFF, 0)),
        out_shape=jax.ShapeDtypeStruct((SEQ_LEN, D_MODEL), jnp.float32),
        input_output_aliases={2: 0},
    )(x, emb, prev)


def kernel(x, emb):
    sc_full = _sc_add(x, emb)
    return _tc_finish(x, emb, sc_full)


# speedup vs baseline: 1.2460x; 1.0127x over previous
import functools

import jax
import jax.numpy as jnp
from jax import lax
from jax.experimental import pallas as pl
from jax.experimental.pallas import tpu as pltpu
from jax.experimental.pallas import tpu_sc as plsc

SEQ_LEN = 8192
D_MODEL = 1024
LANES = 16
NUM_WORKERS = 32
ROWS_PER_WORKER = SEQ_LEN // NUM_WORKERS
CHUNK_ROWS = 16
NUM_CHUNKS = ROWS_PER_WORKER // CHUNK_ROWS
NBUF = 2

_mesh = plsc.VectorSubcoreMesh(core_axis_name="c", subcore_axis_name="s")

_CHUNK = (CHUNK_ROWS, D_MODEL)
_scratch = (
    [pltpu.VMEM(_CHUNK, jnp.float32) for _ in range(3 * NBUF)]
    + [pltpu.SemaphoreType.DMA for _ in range(3 * NBUF)]
)


@functools.partial(
    pl.kernel,
    mesh=_mesh,
    out_type=jax.ShapeDtypeStruct((SEQ_LEN, D_MODEL), jnp.float32),
    scratch_types=_scratch,
)
def _sc_add(x_hbm, emb_hbm, out_hbm, *scratch):
    bufs = scratch[: 3 * NBUF]
    sems = scratch[3 * NBUF :]
    xbufs, ebufs, obufs = bufs[:NBUF], bufs[NBUF : 2 * NBUF], bufs[2 * NBUF :]
    sxs, ses, sos = sems[:NBUF], sems[NBUF : 2 * NBUF], sems[2 * NBUF :]

    wid = lax.axis_index("c") * 16 + lax.axis_index("s")
    base = wid * ROWS_PER_WORKER

    def rows_at(ci):
        return pl.ds(base + ci * CHUNK_ROWS, CHUNK_ROWS)

    def start_gather(ci, b):
        pltpu.async_copy(x_hbm.at[rows_at(ci), :], xbufs[b], sxs[b])
        pltpu.async_copy(emb_hbm.at[rows_at(ci), :], ebufs[b], ses[b])

    def wait_gather(b):
        pltpu.make_async_copy(x_hbm.at[rows_at(0), :], xbufs[b], sxs[b]).wait()
        pltpu.make_async_copy(emb_hbm.at[rows_at(0), :], ebufs[b], ses[b]).wait()

    def wait_scatter(b):
        pltpu.make_async_copy(obufs[b], out_hbm.at[rows_at(0), :], sos[b]).wait()

    for b in range(NBUF - 1):
        start_gather(b, b)

    def outer(g, carry):
        for b in range(NBUF):
            ci = NBUF * g + b

            @pl.when(ci + NBUF - 1 < NUM_CHUNKS)
            def _():
                start_gather(ci + NBUF - 1, (b + NBUF - 1) % NBUF)

            wait_gather(b)

            @pl.when(ci >= NBUF)
            def _():
                wait_scatter(b)

            xbuf, ebuf, obuf = xbufs[b], ebufs[b], obufs[b]

            def row_body(r, rcarry):
                for j in range(D_MODEL // LANES):
                    sl = pl.ds(j * LANES, LANES)
                    obuf[r, sl] = xbuf[r, sl] + ebuf[r, sl]
                return rcarry

            lax.fori_loop(0, CHUNK_ROWS, row_body, 0)
            pltpu.async_copy(obuf, out_hbm.at[rows_at(ci), :], sos[b])
        return carry

    lax.fori_loop(0, NUM_CHUNKS // NBUF, outer, 0)
    for b in range(NBUF):
        wait_scatter(b)


def kernel(x, emb):
    return _sc_add(x, emb)
